# Initial kernel scaffold; baseline (speedup 1.0000x reference)
import jax
import jax.numpy as jnp
from jax.experimental import pallas as pl


def kernel(token_embedding, pos_table):
    raise NotImplementedError("write your pallas kernel here")


# speedup vs baseline: 1.6964x; 1.6964x over previous
import jax
import jax.numpy as jnp
from jax.experimental import pallas as pl

BS = 512


def _body(tok_ref, pos_ref, out_ref):
    out_ref[...] = tok_ref[...] + pos_ref[...][None, :, :]


def kernel(token_embedding, pos_table):
    B, S, E = token_embedding.shape
    grid = (S // BS, B)
    return pl.pallas_call(
        _body,
        grid=grid,
        in_specs=[
            pl.BlockSpec((1, BS, E), lambda i, b: (b, i, 0)),
            pl.BlockSpec((BS, E), lambda i, b: (i, 0)),
        ],
        out_specs=pl.BlockSpec((1, BS, E), lambda i, b: (b, i, 0)),
        out_shape=jax.ShapeDtypeStruct((B, S, E), token_embedding.dtype),
    )(token_embedding, pos_table)
